# Initial kernel scaffold; baseline (speedup 1.0000x reference)
import jax
import jax.numpy as jnp
from jax.experimental import pallas as pl


def kernel(token_seq, hour_seq, poi_table, hour_table, fc_w, fc_b):
    raise NotImplementedError("write your pallas kernel here")


# speedup vs baseline: 1.8903x; 1.8903x over previous
import functools

import jax
import jax.numpy as jnp
from jax import lax
from jax.experimental import pallas as pl
from jax.experimental.pallas import tpu as pltpu
from jax.experimental.pallas import tpu_sc as plsc

B, L = 4096, 200
E = 64
H = E // 4
NUM_HOURS = 24 + 1
HP = 32

NC, NS = 2, 16
NW = NC * NS

N_TOK = B * L
ROWS_PER_W = N_TOK // NW
CHUNK = 1024
N_CHUNKS = ROWS_PER_W // CHUNK

TC_BLK = 2048
N_BLKS = N_TOK // TC_BLK


def _sc_gather_body(idx_hbm, table_hbm, out_hbm, idx_v, rows_v, sem):
    wid = lax.axis_index("s") * NC + lax.axis_index("c")
    base = wid * ROWS_PER_W

    def step(i, carry):
        off = base + i * CHUNK
        pltpu.sync_copy(idx_hbm.at[pl.ds(off, CHUNK)], idx_v)
        pltpu.async_copy(table_hbm.at[idx_v], rows_v, sem).wait()
        pltpu.sync_copy(rows_v, out_hbm.at[pl.ds(off, CHUNK)])
        return carry

    lax.fori_loop(0, N_CHUNKS, step, 0)


def _sc_gather(token_flat, poi_table):
    mesh = plsc.VectorSubcoreMesh(
        core_axis_name="c", subcore_axis_name="s",
        num_cores=NC, num_subcores=NS,
    )
    kern = pl.kernel(
        _sc_gather_body,
        out_type=jax.ShapeDtypeStruct((N_TOK, E), jnp.float32),
        mesh=mesh,
        compiler_params=pltpu.CompilerParams(use_tc_tiling_on_sc=False),
        scratch_types=[
            pltpu.VMEM((CHUNK,), jnp.int32),
            pltpu.VMEM((CHUNK, E), jnp.float32),
            pltpu.SemaphoreType.DMA,
        ],
    )
    return kern(token_flat, poi_table)


def _tc_body(g_ref, hid_ref, htab_ref, w1_ref, w2_ref, b_ref, out_ref):
    g = g_ref[...]
    hid = hid_ref[0, 0, :]
    oh = (hid[:, None] == lax.broadcasted_iota(jnp.int32, (TC_BLK, HP), 1))
    oh = oh.astype(jnp.float32)
    hemb = jnp.dot(oh, htab_ref[...], preferred_element_type=jnp.float32)
    y = lax.dot_general(g, w1_ref[...], (((1,), (1,)), ((), ())),
                        preferred_element_type=jnp.float32)
    y += lax.dot_general(hemb, w2_ref[...], (((1,), (1,)), ((), ())),
                         preferred_element_type=jnp.float32)
    out_ref[...] = jnp.tanh(y + b_ref[...])


def _tc_fused(gathered, hour_blk, htab_pad, w1, w2, fc_b):
    return pl.pallas_call(
        _tc_body,
        grid=(N_BLKS,),
        in_specs=[
            pl.BlockSpec((TC_BLK, E), lambda i: (i, 0)),
            pl.BlockSpec((1, 1, TC_BLK), lambda i: (i, 0, 0)),
            pl.BlockSpec((HP, H), lambda i: (0, 0)),
            pl.BlockSpec((E, E), lambda i: (0, 0)),
            pl.BlockSpec((E, H), lambda i: (0, 0)),
            pl.BlockSpec((1, E), lambda i: (0, 0)),
        ],
        out_specs=pl.BlockSpec((TC_BLK, E), lambda i: (i, 0)),
        out_shape=jax.ShapeDtypeStruct((N_TOK, E), jnp.float32),
    )(gathered, hour_blk, htab_pad, w1, w2, fc_b)


@jax.jit
def kernel(token_seq, hour_seq, poi_table, hour_table, fc_w, fc_b):
    token_flat = token_seq.reshape(N_TOK).astype(jnp.int32)
    gathered = _sc_gather(token_flat, poi_table)

    hour_blk = hour_seq.reshape(N_BLKS, 1, TC_BLK).astype(jnp.int32)
    htab_pad = jnp.zeros((HP, H), jnp.float32).at[:NUM_HOURS].set(hour_table)
    w1 = fc_w[:, :E]
    w2 = fc_w[:, E:]
    out = _tc_fused(gathered, hour_blk, htab_pad, w1, w2,
                    fc_b.reshape(1, E))
    return out.reshape(B, L, E)


# speedup vs baseline: 3.4702x; 1.8358x over previous
import jax
import jax.numpy as jnp
from jax import lax
from jax.experimental import pallas as pl
from jax.experimental.pallas import tpu as pltpu
from jax.experimental.pallas import tpu_sc as plsc

B, L = 4096, 200
E = 64
H = E // 4
P = 1000000 + 1
NUM_HOURS = 24 + 1
HP = 32

NC, NS = 2, 16
NW = NC * NS

N_TOK = B * L
ROWS_PER_W = N_TOK // NW
CHUNK = 1024
N_CHUNKS = ROWS_PER_W // CHUNK

PROJ_BLK = 4096
N_PROJ_BLKS = -(-P // PROJ_BLK)
P_PAD = N_PROJ_BLKS * PROJ_BLK

TC_BLK = 2048
N_BLKS = N_TOK // TC_BLK


def _proj_body(poi_ref, w1_ref, htab_ref, w2_ref, b_ref, out_ref, hp_ref):
    gt = poi_ref[...]
    proj = lax.dot_general(gt, w1_ref[...], (((0,), (1,)), ((), ())),
                           preferred_element_type=jnp.float32)
    out_ref[...] = jnp.concatenate([proj, jnp.zeros_like(proj)], axis=1)
    hp_ref[...] = lax.dot_general(
        htab_ref[...], w2_ref[...], (((1,), (1,)), ((), ())),
        preferred_element_type=jnp.float32) + b_ref[...]


def _tc_project(poi_table, w1, htab_pad, w2, fc_b):
    return pl.pallas_call(
        _proj_body,
        grid=(N_PROJ_BLKS,),
        in_specs=[
            pl.BlockSpec((E, PROJ_BLK), lambda i: (0, i)),
            pl.BlockSpec((E, E), lambda i: (0, 0)),
            pl.BlockSpec((HP, H), lambda i: (0, 0)),
            pl.BlockSpec((E, H), lambda i: (0, 0)),
            pl.BlockSpec((1, E), lambda i: (0, 0)),
        ],
        out_specs=[
            pl.BlockSpec((PROJ_BLK, 2 * E), lambda i: (i, 0)),
            pl.BlockSpec((HP, E), lambda i: (0, 0)),
        ],
        out_shape=[
            jax.ShapeDtypeStruct((P_PAD, 2 * E), jnp.float32),
            jax.ShapeDtypeStruct((HP, E), jnp.float32),
        ],
    )(poi_table, w1, htab_pad, w2, fc_b)


def _sc_gather_body(idx_hbm, table_hbm, out_hbm, idx_v, rows_v, sem):
    wid = lax.axis_index("s") * NC + lax.axis_index("c")
    base = wid * ROWS_PER_W

    def step(i, carry):
        off = base + i * CHUNK
        pltpu.sync_copy(idx_hbm.at[pl.ds(off, CHUNK)], idx_v)
        pltpu.async_copy(table_hbm.at[idx_v], rows_v, sem).wait()
        pltpu.sync_copy(rows_v, out_hbm.at[pl.ds(off, CHUNK), pl.ds(0, E)])
        return carry

    lax.fori_loop(0, N_CHUNKS, step, 0)


def _sc_gather(token2, table_lin):
    mesh = plsc.VectorSubcoreMesh(
        core_axis_name="c", subcore_axis_name="s",
        num_cores=NC, num_subcores=NS,
    )
    kern = pl.kernel(
        _sc_gather_body,
        out_type=jax.ShapeDtypeStruct((N_TOK, 2 * E), jnp.float32),
        mesh=mesh,
        compiler_params=pltpu.CompilerParams(use_tc_tiling_on_sc=False),
        scratch_types=[
            pltpu.VMEM((CHUNK,), jnp.int32),
            pltpu.VMEM((CHUNK, E), jnp.float32),
            pltpu.SemaphoreType.DMA,
        ],
    )
    return kern(token2, table_lin)


def _fin_body(g_ref, hid_ref, hp_ref, out_ref):
    g = g_ref[:, :E]
    hid = hid_ref[0, 0, :]
    oh = (hid[:, None] == lax.broadcasted_iota(jnp.int32, (TC_BLK, HP), 1))
    oh = oh.astype(jnp.float32)
    hc = jnp.dot(oh, hp_ref[...], preferred_element_type=jnp.float32)
    out_ref[0] = jnp.tanh(g + hc).T


BPB = B // TC_BLK


def _tc_finish(gathered, hour_blk, hproj):
    return pl.pallas_call(
        _fin_body,
        grid=(L, BPB),
        in_specs=[
            pl.BlockSpec((TC_BLK, 2 * E), lambda i, j: (i * BPB + j, 0)),
            pl.BlockSpec((1, 1, TC_BLK), lambda i, j: (i * BPB + j, 0, 0)),
            pl.BlockSpec((HP, E), lambda i, j: (0, 0)),
        ],
        out_specs=pl.BlockSpec((1, E, TC_BLK), lambda i, j: (i, 0, j)),
        out_shape=jax.ShapeDtypeStruct((L, E, B), jnp.float32),
    )(gathered, hour_blk, hproj)


@jax.jit
def kernel(token_seq, hour_seq, poi_table, hour_table, fc_w, fc_b):
    htab_pad = jnp.zeros((HP, H), jnp.float32).at[:NUM_HOURS].set(hour_table)
    proj, hproj = _tc_project(poi_table.T, fc_w[:, :E], htab_pad, fc_w[:, E:],
                              fc_b.reshape(1, E))
    table_lin = proj.reshape(2 * P_PAD, E)

    token2 = (token_seq.T.reshape(N_TOK) * 2).astype(jnp.int32)
    gathered = _sc_gather(token2, table_lin)

    hour_blk = hour_seq.T.reshape(N_BLKS, 1, TC_BLK).astype(jnp.int32)
    out = _tc_finish(gathered, hour_blk, hproj)
    return out.transpose(2, 0, 1)


# speedup vs baseline: 3.8449x; 1.1080x over previous
import jax
import jax.numpy as jnp
from jax import lax
from jax.experimental import pallas as pl
from jax.experimental.pallas import tpu as pltpu
from jax.experimental.pallas import tpu_sc as plsc

B, L = 4096, 200
E = 64
H = E // 4
P = 1000000 + 1
NUM_HOURS = 24 + 1
HP = 32

NC, NS = 2, 16
NW = NC * NS

N_TOK = B * L
ROWS_PER_W = N_TOK // NW
CHUNK = 512

PROJ_BLK = 2048
N_PROJ_BLKS = 245
P_HALF = N_PROJ_BLKS * PROJ_BLK

HB = B // 2


def _proj_body(pa_ref, pb_ref, w1_ref, htab_ref, w2_ref, b_ref,
               out_ref, hp_ref):
    dn = (((0,), (1,)), ((), ()))
    pa = lax.dot_general(pa_ref[...], w1_ref[...], dn,
                         preferred_element_type=jnp.float32)
    pb = lax.dot_general(pb_ref[...], w1_ref[...], dn,
                         preferred_element_type=jnp.float32)
    out_ref[...] = jnp.concatenate([pa, pb], axis=1)
    hp_ref[...] = lax.dot_general(
        htab_ref[...], w2_ref[...], (((1,), (1,)), ((), ())),
        preferred_element_type=jnp.float32) + b_ref[...]


def _tc_project(poi_t, w1, htab_pad, w2, fc_b):
    return pl.pallas_call(
        _proj_body,
        grid=(N_PROJ_BLKS,),
        in_specs=[
            pl.BlockSpec((E, PROJ_BLK), lambda i: (0, i)),
            pl.BlockSpec((E, PROJ_BLK),
                         lambda i: (0, jnp.minimum(i + N_PROJ_BLKS,
                                                   2 * N_PROJ_BLKS - 2))),
            pl.BlockSpec((E, E), lambda i: (0, 0)),
            pl.BlockSpec((HP, H), lambda i: (0, 0)),
            pl.BlockSpec((E, H), lambda i: (0, 0)),
            pl.BlockSpec((1, E), lambda i: (0, 0)),
        ],
        out_specs=[
            pl.BlockSpec((PROJ_BLK, 2 * E), lambda i: (i, 0)),
            pl.BlockSpec((HP, E), lambda i: (0, 0)),
        ],
        out_shape=[
            jax.ShapeDtypeStruct((P_HALF, 2 * E), jnp.float32),
            jax.ShapeDtypeStruct((HP, E), jnp.float32),
        ],
    )(poi_t, poi_t, w1, htab_pad, w2, fc_b)


N_CHUNKS = ROWS_PER_W // CHUNK


def _sc_gather_body(idx_hbm, table_hbm, out_hbm, idx_v, rows_v, sem):
    wid = lax.axis_index("s") * NC + lax.axis_index("c")
    base = wid * ROWS_PER_W

    def step(i, carry):
        off = base + i * CHUNK
        pltpu.sync_copy(idx_hbm.at[pl.ds(off, CHUNK)], idx_v)
        pltpu.async_copy(table_hbm.at[idx_v], rows_v, sem).wait()
        pltpu.sync_copy(rows_v, out_hbm.at[pl.ds(off, CHUNK), pl.ds(0, E)])
        return carry

    lax.fori_loop(0, N_CHUNKS, step, 0)


def _sc_gather(token_flat, table_lin):
    mesh = plsc.VectorSubcoreMesh(
        core_axis_name="c", subcore_axis_name="s",
        num_cores=NC, num_subcores=NS,
    )
    kern = pl.kernel(
        _sc_gather_body,
        out_type=jax.ShapeDtypeStruct((N_TOK, 2 * E), jnp.float32),
        mesh=mesh,
        compiler_params=pltpu.CompilerParams(use_tc_tiling_on_sc=False),
        scratch_types=[
            pltpu.VMEM((CHUNK,), jnp.int32),
            pltpu.VMEM((CHUNK, E), jnp.float32),
            pltpu.SemaphoreType.DMA,
        ],
    )
    return kern(token_flat, table_lin)


def _fin_body(g_ref, hid_ref, hp_ref, out_ref):
    g = g_ref[:, :E]
    hid = hid_ref[0, 0, :]
    oh = (hid[:, None] == lax.broadcasted_iota(jnp.int32, (B, HP), 1))
    oh = oh.astype(jnp.float32)
    hc = jnp.dot(oh, hp_ref[...], preferred_element_type=jnp.float32)
    out_ref[0] = jnp.tanh(g + hc).T


def _tc_finish(gathered, hour_blk, hproj):
    return pl.pallas_call(
        _fin_body,
        grid=(L,),
        in_specs=[
            pl.BlockSpec((B, 2 * E), lambda i: (i, 0)),
            pl.BlockSpec((1, 1, B), lambda i: (i, 0, 0)),
            pl.BlockSpec((HP, E), lambda i: (0, 0)),
        ],
        out_specs=pl.BlockSpec((1, E, B), lambda i: (i, 0, 0)),
        out_shape=jax.ShapeDtypeStruct((L, E, B), jnp.float32),
    )(gathered, hour_blk, hproj)


@jax.jit
def kernel(token_seq, hour_seq, poi_table, hour_table, fc_w, fc_b):
    htab_pad = jnp.zeros((HP, H), jnp.float32).at[:NUM_HOURS].set(hour_table)
    proj, hproj = _tc_project(poi_table.T, fc_w[:, :E], htab_pad, fc_w[:, E:],
                              fc_b.reshape(1, E))
    table_lin = proj.reshape(2 * P_HALF, E)

    tok = token_seq.T.reshape(N_TOK).astype(jnp.int32)
    tok2 = jnp.where(tok < P_HALF, 2 * tok, 2 * (tok - P_HALF) + 1)
    gathered = _sc_gather(tok2, table_lin)

    hour_blk = hour_seq.T.reshape(L, 1, B).astype(jnp.int32)
    out = _tc_finish(gathered, hour_blk, hproj)
    return out.transpose(2, 0, 1)


# speedup vs baseline: 4.0852x; 1.0625x over previous
import jax
import jax.numpy as jnp
from jax import lax
from jax.experimental import pallas as pl
from jax.experimental.pallas import tpu as pltpu
from jax.experimental.pallas import tpu_sc as plsc

B, L = 4096, 200
E = 64
H = E // 4
P = 1000000 + 1
NUM_HOURS = 24 + 1
HP = 32

NC, NS = 2, 16
NW = NC * NS

N_TOK = B * L
HB = B // 2

PROJ_BLK = 4096
N_PROJ_BLKS = -(-P // (2 * PROJ_BLK))
P_HALF = N_PROJ_BLKS * PROJ_BLK
TBL_MINOR_BLKS = -(-P // PROJ_BLK)

CHUNK = 512
OUT_ROWS_PER_W = (N_TOK // 2) // NW
N_OCHUNKS = OUT_ROWS_PER_W // CHUNK


def _proj_body(pa_ref, pb_ref, w1_ref, htab_ref, w2_ref, b_ref,
               out_ref, hp_ref):
    dn = (((0,), (1,)), ((), ()))
    pa = lax.dot_general(pa_ref[...], w1_ref[...], dn,
                         preferred_element_type=jnp.float32)
    pb = lax.dot_general(pb_ref[...], w1_ref[...], dn,
                         preferred_element_type=jnp.float32)
    out_ref[...] = jnp.concatenate([pa, pb], axis=1)
    hp_ref[...] = lax.dot_general(
        htab_ref[...], w2_ref[...], (((1,), (1,)), ((), ())),
        preferred_element_type=jnp.float32) + b_ref[...]


def _tc_project(poi_t, w1, htab_pad, w2, fc_b):
    return pl.pallas_call(
        _proj_body,
        grid=(N_PROJ_BLKS,),
        in_specs=[
            pl.BlockSpec((E, PROJ_BLK), lambda i: (0, i)),
            pl.BlockSpec((E, PROJ_BLK),
                         lambda i: (0, jnp.minimum(i + N_PROJ_BLKS,
                                                   TBL_MINOR_BLKS - 1))),
            pl.BlockSpec((E, E), lambda i: (0, 0)),
            pl.BlockSpec((HP, H), lambda i: (0, 0)),
            pl.BlockSpec((E, H), lambda i: (0, 0)),
            pl.BlockSpec((1, E), lambda i: (0, 0)),
        ],
        out_specs=[
            pl.BlockSpec((PROJ_BLK, 2 * E), lambda i: (i, 0)),
            pl.BlockSpec((HP, E), lambda i: (0, 0)),
        ],
        out_shape=[
            jax.ShapeDtypeStruct((P_HALF, 2 * E), jnp.float32),
            jax.ShapeDtypeStruct((HP, E), jnp.float32),
        ],
    )(poi_t, poi_t, w1, htab_pad, w2, fc_b)


def _sc_gather_body(idx_hbm, table_hbm, out_hbm,
                    idx_a, idx_b, rows_a, rows_b, sem):
    wid = lax.axis_index("s") * NC + lax.axis_index("c")
    base = wid * OUT_ROWS_PER_W

    def step(i, carry):
        r0 = base + i * CHUNK
        off_a = r0 + (r0 // HB) * HB
        off_b = off_a + HB
        pltpu.sync_copy(idx_hbm.at[pl.ds(off_a, CHUNK)], idx_a)
        pltpu.sync_copy(idx_hbm.at[pl.ds(off_b, CHUNK)], idx_b)
        pltpu.async_copy(table_hbm.at[idx_a], rows_a, sem).wait()
        pltpu.async_copy(table_hbm.at[idx_b], rows_b, sem).wait()
        pltpu.sync_copy(rows_a, out_hbm.at[pl.ds(r0, CHUNK), pl.ds(0, E)])
        pltpu.sync_copy(rows_b, out_hbm.at[pl.ds(r0, CHUNK), pl.ds(E, E)])
        return carry

    lax.fori_loop(0, N_OCHUNKS, step, 0)


def _sc_gather(token_flat, table_lin):
    mesh = plsc.VectorSubcoreMesh(
        core_axis_name="c", subcore_axis_name="s",
        num_cores=NC, num_subcores=NS,
    )
    kern = pl.kernel(
        _sc_gather_body,
        out_type=jax.ShapeDtypeStruct((N_TOK // 2, 2 * E), jnp.float32),
        mesh=mesh,
        compiler_params=pltpu.CompilerParams(use_tc_tiling_on_sc=False),
        scratch_types=[
            pltpu.VMEM((CHUNK,), jnp.int32),
            pltpu.VMEM((CHUNK,), jnp.int32),
            pltpu.VMEM((CHUNK, E), jnp.float32),
            pltpu.VMEM((CHUNK, E), jnp.float32),
            pltpu.SemaphoreType.DMA,
        ],
    )
    return kern(token_flat, table_lin)


def _fin_body(g_ref, hid_ref, hp_ref, out_ref):
    g = g_ref[...]
    hid = hid_ref[0, 0, :]
    iota = lax.broadcasted_iota(jnp.int32, (HB, HP), 1)
    oha = (hid[:HB, None] == iota).astype(jnp.float32)
    ohb = (hid[HB:, None] == iota).astype(jnp.float32)
    hp = hp_ref[...]
    hca = jnp.dot(oha, hp, preferred_element_type=jnp.float32)
    hcb = jnp.dot(ohb, hp, preferred_element_type=jnp.float32)
    ya = jnp.tanh(g[:, :E] + hca).T
    yb = jnp.tanh(g[:, E:] + hcb).T
    out_ref[0] = jnp.concatenate([ya, yb], axis=1)


def _tc_finish(gathered, hour_blk, hproj):
    return pl.pallas_call(
        _fin_body,
        grid=(L,),
        in_specs=[
            pl.BlockSpec((HB, 2 * E), lambda i: (i, 0)),
            pl.BlockSpec((1, 1, B), lambda i: (i, 0, 0)),
            pl.BlockSpec((HP, E), lambda i: (0, 0)),
        ],
        out_specs=pl.BlockSpec((1, E, B), lambda i: (i, 0, 0)),
        out_shape=jax.ShapeDtypeStruct((L, E, B), jnp.float32),
    )(gathered, hour_blk, hproj)


@jax.jit
def kernel(token_seq, hour_seq, poi_table, hour_table, fc_w, fc_b):
    htab_pad = jnp.zeros((HP, H), jnp.float32).at[:NUM_HOURS].set(hour_table)
    proj, hproj = _tc_project(poi_table.T, fc_w[:, :E], htab_pad, fc_w[:, E:],
                              fc_b.reshape(1, E))
    table_lin = proj.reshape(2 * P_HALF, E)

    tok = token_seq.T.reshape(N_TOK).astype(jnp.int32)
    tok2 = jnp.where(tok < P_HALF, 2 * tok, 2 * (tok - P_HALF) + 1)
    gathered = _sc_gather(tok2, table_lin)

    hour_blk = hour_seq.T.reshape(L, 1, B).astype(jnp.int32)
    out = _tc_finish(gathered, hour_blk, hproj)
    return out.transpose(2, 0, 1)


# speedup vs baseline: 4.5267x; 1.1081x over previous
import jax
import jax.numpy as jnp
from jax import lax
from jax.experimental import pallas as pl
from jax.experimental.pallas import tpu as pltpu
from jax.experimental.pallas import tpu_sc as plsc

B, L = 4096, 200
E = 64
H = E // 4
P = 1000000 + 1
NUM_HOURS = 24 + 1
HP = 32

NC, NS = 2, 16
NW = NC * NS

N_TOK = B * L
HB = B // 2

PROJ_BLK = 4096
N_PROJ_BLKS = -(-P // (2 * PROJ_BLK))
P_HALF = N_PROJ_BLKS * PROJ_BLK
TBL_MINOR_BLKS = -(-P // PROJ_BLK)

CHUNK = 512
OUT_ROWS_PER_W = (N_TOK // 2) // NW
N_OCHUNKS = OUT_ROWS_PER_W // CHUNK


def _proj_body(pa_ref, pb_ref, w1_ref, htab_ref, w2_ref, b_ref,
               out_ref, hp_ref):
    dn = (((0,), (1,)), ((), ()))
    pa = lax.dot_general(pa_ref[...], w1_ref[...], dn,
                         preferred_element_type=jnp.float32)
    pb = lax.dot_general(pb_ref[...], w1_ref[...], dn,
                         preferred_element_type=jnp.float32)
    out_ref[...] = jnp.concatenate([pa, pb], axis=1)
    hp_ref[...] = lax.dot_general(
        htab_ref[...], w2_ref[...], (((1,), (1,)), ((), ())),
        preferred_element_type=jnp.float32) + b_ref[...]


def _tc_project(poi_t, w1, htab_pad, w2, fc_b):
    return pl.pallas_call(
        _proj_body,
        grid=(N_PROJ_BLKS,),
        in_specs=[
            pl.BlockSpec((E, PROJ_BLK), lambda i: (0, i)),
            pl.BlockSpec((E, PROJ_BLK),
                         lambda i: (0, jnp.minimum(i + N_PROJ_BLKS,
                                                   TBL_MINOR_BLKS - 1))),
            pl.BlockSpec((E, E), lambda i: (0, 0)),
            pl.BlockSpec((HP, H), lambda i: (0, 0)),
            pl.BlockSpec((E, H), lambda i: (0, 0)),
            pl.BlockSpec((1, E), lambda i: (0, 0)),
        ],
        out_specs=[
            pl.BlockSpec((PROJ_BLK, 2 * E), lambda i: (i, 0)),
            pl.BlockSpec((HP, E), lambda i: (0, 0)),
        ],
        out_shape=[
            jax.ShapeDtypeStruct((P_HALF, 2 * E), jnp.float32),
            jax.ShapeDtypeStruct((HP, E), jnp.float32),
        ],
        compiler_params=pltpu.CompilerParams(
            dimension_semantics=("parallel",)),
    )(poi_t, poi_t, w1, htab_pad, w2, fc_b)


def _sc_gather_body(idx_hbm, table_hbm, out_hbm,
                    idx_a, idx_b, rows_a, rows_b, sem):
    wid = lax.axis_index("s") * NC + lax.axis_index("c")
    base = wid * OUT_ROWS_PER_W

    def step(i, carry):
        r0 = base + i * CHUNK
        off_a = r0 + (r0 // HB) * HB
        off_b = off_a + HB
        pltpu.sync_copy(idx_hbm.at[pl.ds(off_a, CHUNK)], idx_a)
        pltpu.sync_copy(idx_hbm.at[pl.ds(off_b, CHUNK)], idx_b)
        pltpu.async_copy(table_hbm.at[idx_a], rows_a, sem).wait()
        pltpu.async_copy(table_hbm.at[idx_b], rows_b, sem).wait()
        pltpu.sync_copy(rows_a, out_hbm.at[pl.ds(r0, CHUNK), pl.ds(0, E)])
        pltpu.sync_copy(rows_b, out_hbm.at[pl.ds(r0, CHUNK), pl.ds(E, E)])
        return carry

    lax.fori_loop(0, N_OCHUNKS, step, 0)


def _sc_gather(token_flat, table_lin):
    mesh = plsc.VectorSubcoreMesh(
        core_axis_name="c", subcore_axis_name="s",
        num_cores=NC, num_subcores=NS,
    )
    kern = pl.kernel(
        _sc_gather_body,
        out_type=jax.ShapeDtypeStruct((N_TOK // 2, 2 * E), jnp.float32),
        mesh=mesh,
        compiler_params=pltpu.CompilerParams(use_tc_tiling_on_sc=False),
        scratch_types=[
            pltpu.VMEM((CHUNK,), jnp.int32),
            pltpu.VMEM((CHUNK,), jnp.int32),
            pltpu.VMEM((CHUNK, E), jnp.float32),
            pltpu.VMEM((CHUNK, E), jnp.float32),
            pltpu.SemaphoreType.DMA,
        ],
    )
    return kern(token_flat, table_lin)


def _fin_body(g_ref, hid_ref, hp_ref, out_ref):
    g = g_ref[...]
    hid = hid_ref[0, 0, :]
    iota = lax.broadcasted_iota(jnp.int32, (HP, HB), 0)
    oha = (hid[None, :HB] == iota).astype(jnp.float32)
    ohb = (hid[None, HB:] == iota).astype(jnp.float32)
    hp = hp_ref[...]
    dn = (((0,), (0,)), ((), ()))
    hca = lax.dot_general(hp, oha, dn,
                          preferred_element_type=jnp.float32)
    hcb = lax.dot_general(hp, ohb, dn,
                          preferred_element_type=jnp.float32)
    ya = jnp.tanh(g[:, :E].T + hca)
    yb = jnp.tanh(g[:, E:].T + hcb)
    out_ref[0] = jnp.concatenate([ya, yb], axis=1)


def _tc_finish(gathered, hour_blk, hproj):
    return pl.pallas_call(
        _fin_body,
        grid=(L,),
        in_specs=[
            pl.BlockSpec((HB, 2 * E), lambda i: (i, 0)),
            pl.BlockSpec((1, 1, B), lambda i: (i, 0, 0)),
            pl.BlockSpec((HP, E), lambda i: (0, 0)),
        ],
        out_specs=pl.BlockSpec((1, E, B), lambda i: (i, 0, 0)),
        out_shape=jax.ShapeDtypeStruct((L, E, B), jnp.float32),
        compiler_params=pltpu.CompilerParams(
            dimension_semantics=("parallel",)),
    )(gathered, hour_blk, hproj)


@jax.jit
def kernel(token_seq, hour_seq, poi_table, hour_table, fc_w, fc_b):
    htab_pad = jnp.zeros((HP, H), jnp.float32).at[:NUM_HOURS].set(hour_table)
    proj, hproj = _tc_project(poi_table.T, fc_w[:, :E], htab_pad, fc_w[:, E:],
                              fc_b.reshape(1, E))
    table_lin = proj.reshape(2 * P_HALF, E)

    tok = token_seq.T.reshape(N_TOK).astype(jnp.int32)
    tok2 = jnp.where(tok < P_HALF, 2 * tok, 2 * (tok - P_HALF) + 1)
    gathered = _sc_gather(tok2, table_lin)

    hour_blk = hour_seq.T.reshape(L, 1, B).astype(jnp.int32)
    out = _tc_finish(gathered, hour_blk, hproj)
    return out.transpose(2, 0, 1)
